# Initial kernel scaffold; baseline (speedup 1.0000x reference)
import jax
import jax.numpy as jnp
from jax.experimental import pallas as pl


def kernel(x, A, B, bias, W_dec, b_dec):
    raise NotImplementedError("write your pallas kernel here")


# speedup vs baseline: 5.1814x; 5.1814x over previous
import jax
import jax.numpy as jnp
from jax import lax
from jax.experimental import pallas as pl
from jax.experimental.pallas import tpu as pltpu
from jax.experimental.pallas import tpu_sc as plsc

N, D, M, R, K = 4096, 1024, 32768, 64, 32
CH = 128
NCH = M // CH
GCOLS = 40
CAND = 128
NW = 32
RPW = N // NW
NEG = float("-inf")


def _k1_body(x_ref, bdec_ref, b_ref, out_ref):
    xc = x_ref[...] - bdec_ref[...]
    out_ref[...] = lax.dot_general(
        xc, b_ref[...], (((1,), (1,)), ((), ())),
        preferred_element_type=jnp.float32)


def _k1(x, b_dec2d, B):
    return pl.pallas_call(
        _k1_body,
        out_shape=jax.ShapeDtypeStruct((N, R), jnp.float32),
    )(x, b_dec2d, B)


BN2 = 128


def _k2_body(int_ref, a_ref, bias_ref, pre_ref, cmax_ref):
    p = lax.dot_general(
        int_ref[...], a_ref[...], (((1,), (1,)), ((), ())),
        preferred_element_type=jnp.float32)
    p = jnp.maximum(p + bias_ref[...], 0.0)
    pre_ref[...] = p
    cmax_ref[...] = jnp.max(p.reshape(BN2, NCH, CH), axis=-1)


def _k2(inter, A, bias2d):
    return pl.pallas_call(
        _k2_body,
        grid=(N // BN2,),
        in_specs=[
            pl.BlockSpec((BN2, R), lambda n: (n, 0)),
            pl.BlockSpec((M, R), lambda n: (0, 0)),
            pl.BlockSpec((1, M), lambda n: (0, 0)),
        ],
        out_specs=[
            pl.BlockSpec((BN2, M), lambda n: (n, 0)),
            pl.BlockSpec((BN2, NCH), lambda n: (n, 0)),
        ],
        out_shape=[
            jax.ShapeDtypeStruct((N, M), jnp.float32),
            jax.ShapeDtypeStruct((N, NCH), jnp.float32),
        ],
    )(inter, A, bias2d)


BN3 = 512


def _k3_body(cmax_ref, gid_ref):
    c = cmax_ref[...]
    n0 = pl.program_id(0) * BN3
    iota2 = lax.broadcasted_iota(jnp.int32, (BN3, NCH), 1)
    rows = n0 + lax.broadcasted_iota(jnp.int32, (BN3, 1), 0)
    ids = []
    m = None
    for _ in range(K):
        m = jnp.max(c, axis=1, keepdims=True)
        amask = c == m
        pos = jnp.min(jnp.where(amask, iota2, NCH), axis=1, keepdims=True)
        ids.append(pos)
        c = jnp.where(iota2 == pos, NEG, c)
    gids = jnp.concatenate(ids, axis=1) + rows * NCH
    tbits = lax.bitcast_convert_type(m, jnp.int32)
    pad = jnp.zeros((BN3, GCOLS - K - 1), jnp.int32)
    gid_ref[...] = jnp.concatenate([gids, tbits, pad], axis=1)


def _k3(cmax):
    return pl.pallas_call(
        _k3_body,
        grid=(N // BN3,),
        in_specs=[pl.BlockSpec((BN3, NCH), lambda n: (n, 0))],
        out_specs=pl.BlockSpec((BN3, GCOLS), lambda n: (n, 0)),
        out_shape=jax.ShapeDtypeStruct((N, GCOLS), jnp.int32),
    )(cmax)


def _k4_body(pre_chunks, gid, cval_out, cidx_out, gid_v, chunks_v,
             cval_v, cidx_v, sem):
    cid = lax.axis_index("c")
    sid = lax.axis_index("s")
    wid = sid * 2 + cid
    base = wid * RPW
    iota16 = lax.iota(jnp.int32, 16)
    neginf = jnp.full((16,), NEG, jnp.float32)

    def row_body(i, carry):
        r = base + i
        pltpu.sync_copy(gid.at[r], gid_v)
        cp = pltpu.async_copy(
            pre_chunks.at[gid_v.at[pl.ds(0, K)]], chunks_v, sem)
        for j in range(CAND // 16):
            cval_v[pl.ds(j * 16, 16)] = neginf
        ti = plsc.load_gather(gid_v, [jnp.full((16,), K, jnp.int32)])
        t = plsc.bitcast(ti, jnp.float32)
        cp.wait()
        rbase = r * M

        def slot_body(s, off):
            cg = plsc.load_gather(gid_v, [jnp.full((16,), s, jnp.int32)])
            gbase = cg * CH - rbase
            row_ref = chunks_v.at[s]
            for j in range(CH // 16):
                v = row_ref[pl.ds(j * 16, 16)]
                mask = v >= t
                minc = jnp.where(mask, 1, 0).astype(jnp.int32)
                incl = plsc.cumsum(minc)
                addr = off + incl - 1
                okm = mask & (addr < CAND)
                plsc.store_scatter(cval_v, [addr], v, mask=okm)
                gx = gbase + (j * 16 + iota16)
                plsc.store_scatter(cidx_v, [addr], gx, mask=okm)
                off = off + plsc.all_reduce_population_count(mask)
            return off

        lax.fori_loop(0, K, slot_body, jnp.zeros((16,), jnp.int32))
        pltpu.sync_copy(cval_v, cval_out.at[r])
        pltpu.sync_copy(cidx_v, cidx_out.at[r])
        return carry

    lax.fori_loop(0, RPW, row_body, 0)


def _build_k4():
    return pl.kernel(
        _k4_body,
        out_type=(jax.ShapeDtypeStruct((N, CAND), jnp.float32),
                  jax.ShapeDtypeStruct((N, CAND), jnp.int32)),
        mesh=plsc.VectorSubcoreMesh(core_axis_name="c", subcore_axis_name="s"),
        compiler_params=pltpu.CompilerParams(needs_layout_passes=False),
        scratch_types=[
            pltpu.VMEM((GCOLS,), jnp.int32),
            pltpu.VMEM((K, CH), jnp.float32),
            pltpu.VMEM((CAND,), jnp.float32),
            pltpu.VMEM((CAND,), jnp.int32),
            pltpu.SemaphoreType.DMA,
        ],
    )


BN5 = 512


def _k5_body(cv_ref, ci_ref, val_ref, idx_ref):
    cv = cv_ref[...]
    ci = ci_ref[...]
    iota2 = lax.broadcasted_iota(jnp.int32, (BN5, CAND), 1)
    vals, idxs = [], []
    for _ in range(K):
        m = jnp.max(cv, axis=1, keepdims=True)
        amask = cv == m
        pos = jnp.min(jnp.where(amask, iota2, CAND), axis=1, keepdims=True)
        pmask = iota2 == pos
        vals.append(jnp.broadcast_to(m, (BN5, 16)))
        idxs.append(jnp.sum(jnp.where(pmask, ci, 0), axis=1, keepdims=True))
        cv = jnp.where(pmask, NEG, cv)
    val_ref[...] = jnp.concatenate(vals, axis=1)
    idx_ref[...] = jnp.concatenate(idxs, axis=1)


def _k5(cval, cidx):
    return pl.pallas_call(
        _k5_body,
        grid=(N // BN5,),
        in_specs=[
            pl.BlockSpec((BN5, CAND), lambda n: (n, 0)),
            pl.BlockSpec((BN5, CAND), lambda n: (n, 0)),
        ],
        out_specs=[
            pl.BlockSpec((BN5, K * 16), lambda n: (n, 0)),
            pl.BlockSpec((BN5, K), lambda n: (n, 0)),
        ],
        out_shape=[
            jax.ShapeDtypeStruct((N, K * 16), jnp.float32),
            jax.ShapeDtypeStruct((N, K), jnp.int32),
        ],
    )(cval, cidx)


def _k6_body(wsplat, indices, wdec, bdec, out, wsp_v, idx_v, wrow_v,
             acc_v, bdec_v, sem):
    cid = lax.axis_index("c")
    sid = lax.axis_index("s")
    wid = sid * 2 + cid
    base = wid * RPW
    pltpu.sync_copy(bdec, bdec_v)

    def row_body(i, carry):
        r = base + i
        pltpu.sync_copy(wsplat.at[r], wsp_v)
        pltpu.sync_copy(indices.at[r], idx_v)
        pltpu.async_copy(wdec.at[idx_v], wrow_v, sem).wait()
        for dg in range(4):
            a = tuple(bdec_v[pl.ds(dg * 256 + q * 16, 16)] for q in range(16))
            for k in range(K):
                w = wsp_v[pl.ds(k * 16, 16)]
                a = tuple(
                    a[q] + w * wrow_v[k, pl.ds(dg * 256 + q * 16, 16)]
                    for q in range(16))
            for q in range(16):
                acc_v[pl.ds(dg * 256 + q * 16, 16)] = a[q]
        pltpu.sync_copy(acc_v, out.at[r])
        return carry

    lax.fori_loop(0, RPW, row_body, 0)


def _build_k6():
    return pl.kernel(
        _k6_body,
        out_type=jax.ShapeDtypeStruct((N, D), jnp.float32),
        mesh=plsc.VectorSubcoreMesh(core_axis_name="c", subcore_axis_name="s"),
        compiler_params=pltpu.CompilerParams(needs_layout_passes=False),
        scratch_types=[
            pltpu.VMEM((K * 16,), jnp.float32),
            pltpu.VMEM((K,), jnp.int32),
            pltpu.VMEM((K, D), jnp.float32),
            pltpu.VMEM((D,), jnp.float32),
            pltpu.VMEM((D,), jnp.float32),
            pltpu.SemaphoreType.DMA,
        ],
    )


_sc_cache = {}


def _sc(name):
    if name not in _sc_cache:
        _sc_cache[name] = {"k4": _build_k4, "k6": _build_k6}[name]()
    return _sc_cache[name]


@jax.jit
def kernel(x, A, B, bias, W_dec, b_dec):
    inter = _k1(x, b_dec.reshape(1, D), B)
    pre, cmax = _k2(inter, A, bias.reshape(1, M))
    gid = _k3(cmax)
    cval, cidx = _sc("k4")(pre.reshape(N * NCH, CH), gid)
    wsplat, indices = _k5(cval, cidx)
    return _sc("k6")(wsplat, indices, W_dec, b_dec)
